# Initial kernel scaffold; baseline (speedup 1.0000x reference)
import jax
import jax.numpy as jnp
from jax.experimental import pallas as pl


def kernel(x, edge_index, batch, Wl0, bl0, Wr0, br0, Wl1, bl1, Wr1, br1, Wl2, bl2, Wr2, br2, Wp1, bp1, Wp2, bp2):
    raise NotImplementedError("write your pallas kernel here")


# speedup vs baseline: 4.6294x; 4.6294x over previous
import functools

import jax
import jax.numpy as jnp
from jax import lax
from jax.experimental import pallas as pl
from jax.experimental.pallas import tpu as pltpu
from jax.experimental.pallas import tpu_sc as plsc

N = 10000
E = 320000
D = 128
HID = 128
OUTD = 64
G = 128

NC = 2
NS = 16
NW = NC * NS
EPT = E // NW
K = 80
NCHUNK = EPT // K
CW = 16
RPT = 632
RPT_LAST = N - 15 * RPT


def _zero_acc(sid, zrow_hbm, acc):
    rbase = pl.multiple_of(sid * RPT, 8)

    @pl.when(sid < NS - 1)
    def _():
        pltpu.sync_copy(zrow_hbm.at[pl.ds(rbase, RPT)],
                        acc.at[pl.ds(rbase, RPT)])

    @pl.when(sid == NS - 1)
    def _():
        pltpu.sync_copy(zrow_hbm.at[pl.ds((NS - 1) * RPT, RPT_LAST)],
                        acc.at[pl.ds((NS - 1) * RPT, RPT_LAST)])


def _copy_out(cid, sid, acc, out_hbm):
    rbase = pl.multiple_of(sid * RPT, 8)
    obase = pl.multiple_of(cid * N + rbase, 8)

    @pl.when(sid < NS - 1)
    def _():
        pltpu.sync_copy(acc.at[pl.ds(rbase, RPT)],
                        out_hbm.at[pl.ds(obase, RPT)])

    @pl.when(sid == NS - 1)
    def _():
        pltpu.sync_copy(
            acc.at[pl.ds((NS - 1) * RPT, RPT_LAST)],
            out_hbm.at[pl.ds(pl.multiple_of(cid * N + (NS - 1) * RPT, 8),
                             RPT_LAST)])


@functools.lru_cache(maxsize=None)
def _make_sc_degree():
    mesh = plsc.VectorSubcoreMesh(core_axis_name="c", subcore_axis_name="s")

    def body(dst_hbm, zrow_hbm, ones_hbm, out_hbm, dst_v, ones_v, acc):
        cid = lax.axis_index("c")
        sid = lax.axis_index("s")
        wid = cid * NS + sid
        _zero_acc(sid, zrow_hbm, acc)
        pltpu.sync_copy(ones_hbm, ones_v)
        plsc.subcore_barrier()

        ebase = wid * EPT

        def step(i, carry):
            b = pl.multiple_of(ebase + i * K, 8)
            pltpu.sync_copy(dst_hbm.at[pl.ds(b, K)], dst_v)
            pltpu.sync_copy(ones_v, acc.at[dst_v], add=True)
            return carry

        lax.fori_loop(0, NCHUNK, step, 0)

        plsc.subcore_barrier()
        _copy_out(cid, sid, acc, out_hbm)

    return pl.kernel(
        body, mesh=mesh,
        out_type=jax.ShapeDtypeStruct((NC * N, D), jnp.float32),
        scratch_types=[
            pltpu.VMEM((K,), jnp.int32),
            pltpu.VMEM((K, D), jnp.float32),
            pltpu.VMEM_SHARED((N, D), jnp.float32),
        ])


@functools.lru_cache(maxsize=None)
def _make_sc_segment_sum(w):
    mesh = plsc.VectorSubcoreMesh(core_axis_name="c", subcore_axis_name="s")

    def body(h_hbm, src_hbm, dst_hbm, zrow_hbm, out_hbm,
             src_v, dst_v, rows_v, acc, sem):
        cid = lax.axis_index("c")
        sid = lax.axis_index("s")
        wid = cid * NS + sid
        _zero_acc(sid, zrow_hbm, acc)
        plsc.subcore_barrier()

        ebase = wid * EPT

        def step(i, carry):
            b = pl.multiple_of(ebase + i * K, 8)
            pltpu.sync_copy(src_hbm.at[pl.ds(b, K)], src_v)
            pltpu.sync_copy(dst_hbm.at[pl.ds(b, K)], dst_v)
            pltpu.async_copy(h_hbm.at[src_v], rows_v, sem).wait()
            pltpu.sync_copy(rows_v, acc.at[dst_v], add=True)
            return carry

        lax.fori_loop(0, NCHUNK, step, 0)

        plsc.subcore_barrier()
        _copy_out(cid, sid, acc, out_hbm)

    return pl.kernel(
        body, mesh=mesh,
        out_type=jax.ShapeDtypeStruct((NC * N, w), jnp.float32),
        scratch_types=[
            pltpu.VMEM((K,), jnp.int32),
            pltpu.VMEM((K,), jnp.int32),
            pltpu.VMEM((K, w), jnp.float32),
            pltpu.VMEM_SHARED((N, w), jnp.float32),
            pltpu.SemaphoreType.DMA,
        ])


RB = 1000
_NB = N // RB


def _tc_layer0(part, cntp, h, Wr, br, Wl, bl):
    def body(p0, p1, c0, c1, h_ref, wr, brr, wl, blr, out_ref, cnt_ref):
        p = p0[...] + p1[...]
        c = (c0[...] + c1[...])[:, :1]
        inv = 1.0 / jnp.maximum(c, 1.0)
        ind = jnp.minimum(c, 1.0)
        agg = jnp.dot(p, wr[...], preferred_element_type=jnp.float32) * inv \
            + brr[...] * ind
        out = agg + jnp.dot(h_ref[...], wl[...],
                            preferred_element_type=jnp.float32) + blr[...]
        nrm = jnp.sqrt(jnp.sum(out * out, axis=1, keepdims=True))
        out = out / jnp.maximum(nrm, 1e-12)
        out_ref[...] = jnp.maximum(out, 0.0)
        cnt_ref[...] = c

    return pl.pallas_call(
        body,
        grid=(_NB,),
        in_specs=[
            pl.BlockSpec((RB, D), lambda i: (i, 0)),
            pl.BlockSpec((RB, D), lambda i: (i + _NB, 0)),
            pl.BlockSpec((RB, D), lambda i: (i, 0)),
            pl.BlockSpec((RB, D), lambda i: (i + _NB, 0)),
            pl.BlockSpec((RB, D), lambda i: (i, 0)),
            pl.BlockSpec((D, D), lambda i: (0, 0)),
            pl.BlockSpec((1, D), lambda i: (0, 0)),
            pl.BlockSpec((D, D), lambda i: (0, 0)),
            pl.BlockSpec((1, D), lambda i: (0, 0)),
        ],
        out_specs=[
            pl.BlockSpec((RB, D), lambda i: (i, 0)),
            pl.BlockSpec((RB, 1), lambda i: (i, 0)),
        ],
        out_shape=[
            jax.ShapeDtypeStruct((N, D), jnp.float32),
            jax.ShapeDtypeStruct((N, 1), jnp.float32),
        ],
    )(part, part, cntp, cntp, h, Wr, br, Wl, bl)


def _tc_layer(part, cnt, h, Wr, br, Wl, bl):
    def body(p0, p1, c_ref, h_ref, wr, brr, wl, blr, out_ref):
        p = p0[...] + p1[...]
        c = c_ref[...]
        inv = 1.0 / jnp.maximum(c, 1.0)
        ind = jnp.minimum(c, 1.0)
        agg = jnp.dot(p, wr[...], preferred_element_type=jnp.float32) * inv \
            + brr[...] * ind
        out = agg + jnp.dot(h_ref[...], wl[...],
                            preferred_element_type=jnp.float32) + blr[...]
        nrm = jnp.sqrt(jnp.sum(out * out, axis=1, keepdims=True))
        out = out / jnp.maximum(nrm, 1e-12)
        out_ref[...] = jnp.maximum(out, 0.0)

    return pl.pallas_call(
        body,
        grid=(_NB,),
        in_specs=[
            pl.BlockSpec((RB, D), lambda i: (i, 0)),
            pl.BlockSpec((RB, D), lambda i: (i + _NB, 0)),
            pl.BlockSpec((RB, 1), lambda i: (i, 0)),
            pl.BlockSpec((RB, D), lambda i: (i, 0)),
            pl.BlockSpec((D, D), lambda i: (0, 0)),
            pl.BlockSpec((1, D), lambda i: (0, 0)),
            pl.BlockSpec((D, D), lambda i: (0, 0)),
            pl.BlockSpec((1, D), lambda i: (0, 0)),
        ],
        out_specs=pl.BlockSpec((RB, D), lambda i: (i, 0)),
        out_shape=jax.ShapeDtypeStruct((N, D), jnp.float32),
    )(part, part, cnt, h, Wr, br, Wl, bl)


def _tc_pool(h, batch2d, Wp1, bp1, Wp2, bp2):
    def body(h_ref, b_ref, w1, b1, w2, b2, out_ref):
        bids = b_ref[...]
        gids = lax.broadcasted_iota(jnp.int32, (G, N), 0)
        onehot = (gids == bids).astype(jnp.float32)
        s = jnp.dot(onehot, h_ref[...], preferred_element_type=jnp.float32)
        c = jnp.sum(onehot, axis=1, keepdims=True)
        pooled = s / jnp.maximum(c, 1.0)
        z = jnp.dot(pooled, w1[...], preferred_element_type=jnp.float32) + b1[...]
        z = jnp.dot(z, w2[...], preferred_element_type=jnp.float32) + b2[...]
        z = z * jnp.minimum(c, 1.0)
        m = jnp.max(z, axis=1, keepdims=True)
        e = z - m
        lse = jnp.log(jnp.sum(jnp.exp(e), axis=1, keepdims=True))
        out_ref[...] = e - lse

    return pl.pallas_call(
        body,
        out_shape=jax.ShapeDtypeStruct((G, OUTD), jnp.float32),
    )(h, batch2d, Wp1, bp1, Wp2, bp2)


def kernel(x, edge_index, batch,
           Wl0, bl0, Wr0, br0,
           Wl1, bl1, Wr1, br1,
           Wl2, bl2, Wr2, br2,
           Wp1, bp1, Wp2, bp2):
    src = edge_index[0]
    dst = edge_index[1]
    zrow = jnp.zeros((N, D), jnp.float32)
    ones = jnp.ones((K, D), jnp.float32)

    cntp = _make_sc_degree()(dst, zrow, ones)
    part0 = _make_sc_segment_sum(D)(x, src, dst, zrow)
    h1, cnt = _tc_layer0(part0, cntp, x,
                         Wr0, br0.reshape(1, D), Wl0, bl0.reshape(1, D))
    part1 = _make_sc_segment_sum(D)(h1, src, dst, zrow)
    h2 = _tc_layer(part1, cnt, h1,
                   Wr1, br1.reshape(1, D), Wl1, bl1.reshape(1, D))
    part2 = _make_sc_segment_sum(D)(h2, src, dst, zrow)
    h3 = _tc_layer(part2, cnt, h2,
                   Wr2, br2.reshape(1, D), Wl2, bl2.reshape(1, D))
    return _tc_pool(h3, batch.reshape(1, N),
                    Wp1, bp1.reshape(1, HID), Wp2, bp2.reshape(1, OUTD))


# speedup vs baseline: 8.8350x; 1.9085x over previous
import functools

import jax
import jax.numpy as jnp
from jax import lax
from jax.experimental import pallas as pl
from jax.experimental.pallas import tpu as pltpu
from jax.experimental.pallas import tpu_sc as plsc

N = 10000
E = 320000
D = 128
HID = 128
OUTD = 64
G = 128

NC = 2
NS = 16
NW = NC * NS
EPT = E // NW
K = 80
NCHUNK = EPT // K
SUP = 25
NSB = NCHUNK // SUP
CW = 16
RPT = 632
RPT_LAST = N - 15 * RPT


def _zero_acc(sid, zrow_hbm, acc):
    rbase = pl.multiple_of(sid * RPT, 8)

    @pl.when(sid < NS - 1)
    def _():
        pltpu.sync_copy(zrow_hbm.at[pl.ds(rbase, RPT)],
                        acc.at[pl.ds(rbase, RPT)])

    @pl.when(sid == NS - 1)
    def _():
        pltpu.sync_copy(zrow_hbm.at[pl.ds((NS - 1) * RPT, RPT_LAST)],
                        acc.at[pl.ds((NS - 1) * RPT, RPT_LAST)])


def _copy_out(cid, sid, acc, out_hbm):
    rbase = pl.multiple_of(sid * RPT, 8)
    obase = pl.multiple_of(cid * N + rbase, 8)

    @pl.when(sid < NS - 1)
    def _():
        pltpu.sync_copy(acc.at[pl.ds(rbase, RPT)],
                        out_hbm.at[pl.ds(obase, RPT)])

    @pl.when(sid == NS - 1)
    def _():
        pltpu.sync_copy(
            acc.at[pl.ds((NS - 1) * RPT, RPT_LAST)],
            out_hbm.at[pl.ds(pl.multiple_of(cid * N + (NS - 1) * RPT, 8),
                             RPT_LAST)])


@functools.lru_cache(maxsize=None)
def _make_sc_degree():
    mesh = plsc.VectorSubcoreMesh(core_axis_name="c", subcore_axis_name="s")

    def body(dst_hbm, zrow_hbm, ones_hbm, out_hbm, dst_v, ones_v, acc):
        cid = lax.axis_index("c")
        sid = lax.axis_index("s")
        wid = cid * NS + sid
        _zero_acc(sid, zrow_hbm, acc)
        pltpu.sync_copy(ones_hbm, ones_v)
        plsc.subcore_barrier()

        ebase = wid * EPT

        def step(i, carry):
            b = pl.multiple_of(ebase + i * K, 8)
            pltpu.sync_copy(dst_hbm.at[pl.ds(b, K)], dst_v)
            pltpu.sync_copy(ones_v, acc.at[dst_v], add=True)
            return carry

        lax.fori_loop(0, NCHUNK, step, 0)

        plsc.subcore_barrier()
        _copy_out(cid, sid, acc, out_hbm)

    return pl.kernel(
        body, mesh=mesh,
        out_type=jax.ShapeDtypeStruct((NC * N, D), jnp.float32),
        scratch_types=[
            pltpu.VMEM((K,), jnp.int32),
            pltpu.VMEM((K, D), jnp.float32),
            pltpu.VMEM_SHARED((N, D), jnp.float32),
        ])


@functools.lru_cache(maxsize=None)
def _make_sc_segment_sum(w):
    mesh = plsc.VectorSubcoreMesh(core_axis_name="c", subcore_axis_name="s")

    def body(h_hbm, src_hbm, dst_hbm, zrow_hbm, out_hbm,
             src_v, dst_v, rows_v, acc, sem0, sem1):
        cid = lax.axis_index("c")
        sid = lax.axis_index("s")
        wid = cid * NS + sid
        _zero_acc(sid, zrow_hbm, acc)
        plsc.subcore_barrier()

        def gather(i, slot, sem):
            pltpu.async_copy(h_hbm.at[src_v.at[i]], rows_v.at[slot], sem)

        def wait_gather(i, slot, sem):
            pltpu.make_async_copy(h_hbm.at[src_v.at[i]], rows_v.at[slot],
                                  sem).wait()

        def scatter(i, slot):
            pltpu.sync_copy(rows_v.at[slot], acc.at[dst_v.at[i]], add=True)

        for sb in range(NSB):
            pltpu.sync_copy(src_hbm.at[wid * NSB + sb], src_v)
            pltpu.sync_copy(dst_hbm.at[wid * NSB + sb], dst_v)
            gather(0, 0, sem0)

            def step(t, carry):
                i1 = 2 * t + 1
                gather(i1, 1, sem1)
                wait_gather(i1 - 1, 0, sem0)
                scatter(i1 - 1, 0)
                gather(i1 + 1, 0, sem0)
                wait_gather(i1, 1, sem1)
                scatter(i1, 1)
                return carry

            lax.fori_loop(0, (SUP - 1) // 2, step, 0)
            wait_gather(SUP - 1, 0, sem0)
            scatter(SUP - 1, 0)

        plsc.subcore_barrier()
        _copy_out(cid, sid, acc, out_hbm)

    return pl.kernel(
        body, mesh=mesh,
        out_type=jax.ShapeDtypeStruct((NC * N, w), jnp.float32),
        scratch_types=[
            pltpu.VMEM((SUP, K), jnp.int32),
            pltpu.VMEM((SUP, K), jnp.int32),
            pltpu.VMEM((2, K, w), jnp.float32),
            pltpu.VMEM_SHARED((N, w), jnp.float32),
            pltpu.SemaphoreType.DMA,
            pltpu.SemaphoreType.DMA,
        ])


RB = 1000
_NB = N // RB


def _tc_layer0(part, cntp, h, Wr, br, Wl, bl):
    def body(p0, p1, c0, c1, h_ref, wr, brr, wl, blr, out_ref, cnt_ref):
        p = p0[...] + p1[...]
        c = (c0[...] + c1[...])[:, :1]
        inv = 1.0 / jnp.maximum(c, 1.0)
        ind = jnp.minimum(c, 1.0)
        agg = jnp.dot(p, wr[...], preferred_element_type=jnp.float32) * inv \
            + brr[...] * ind
        out = agg + jnp.dot(h_ref[...], wl[...],
                            preferred_element_type=jnp.float32) + blr[...]
        nrm = jnp.sqrt(jnp.sum(out * out, axis=1, keepdims=True))
        out = out / jnp.maximum(nrm, 1e-12)
        out_ref[...] = jnp.maximum(out, 0.0)
        cnt_ref[...] = c

    return pl.pallas_call(
        body,
        grid=(_NB,),
        in_specs=[
            pl.BlockSpec((RB, D), lambda i: (i, 0)),
            pl.BlockSpec((RB, D), lambda i: (i + _NB, 0)),
            pl.BlockSpec((RB, D), lambda i: (i, 0)),
            pl.BlockSpec((RB, D), lambda i: (i + _NB, 0)),
            pl.BlockSpec((RB, D), lambda i: (i, 0)),
            pl.BlockSpec((D, D), lambda i: (0, 0)),
            pl.BlockSpec((1, D), lambda i: (0, 0)),
            pl.BlockSpec((D, D), lambda i: (0, 0)),
            pl.BlockSpec((1, D), lambda i: (0, 0)),
        ],
        out_specs=[
            pl.BlockSpec((RB, D), lambda i: (i, 0)),
            pl.BlockSpec((RB, 1), lambda i: (i, 0)),
        ],
        out_shape=[
            jax.ShapeDtypeStruct((N, D), jnp.float32),
            jax.ShapeDtypeStruct((N, 1), jnp.float32),
        ],
    )(part, part, cntp, cntp, h, Wr, br, Wl, bl)


def _tc_layer(part, cnt, h, Wr, br, Wl, bl):
    def body(p0, p1, c_ref, h_ref, wr, brr, wl, blr, out_ref):
        p = p0[...] + p1[...]
        c = c_ref[...]
        inv = 1.0 / jnp.maximum(c, 1.0)
        ind = jnp.minimum(c, 1.0)
        agg = jnp.dot(p, wr[...], preferred_element_type=jnp.float32) * inv \
            + brr[...] * ind
        out = agg + jnp.dot(h_ref[...], wl[...],
                            preferred_element_type=jnp.float32) + blr[...]
        nrm = jnp.sqrt(jnp.sum(out * out, axis=1, keepdims=True))
        out = out / jnp.maximum(nrm, 1e-12)
        out_ref[...] = jnp.maximum(out, 0.0)

    return pl.pallas_call(
        body,
        grid=(_NB,),
        in_specs=[
            pl.BlockSpec((RB, D), lambda i: (i, 0)),
            pl.BlockSpec((RB, D), lambda i: (i + _NB, 0)),
            pl.BlockSpec((RB, 1), lambda i: (i, 0)),
            pl.BlockSpec((RB, D), lambda i: (i, 0)),
            pl.BlockSpec((D, D), lambda i: (0, 0)),
            pl.BlockSpec((1, D), lambda i: (0, 0)),
            pl.BlockSpec((D, D), lambda i: (0, 0)),
            pl.BlockSpec((1, D), lambda i: (0, 0)),
        ],
        out_specs=pl.BlockSpec((RB, D), lambda i: (i, 0)),
        out_shape=jax.ShapeDtypeStruct((N, D), jnp.float32),
    )(part, part, cnt, h, Wr, br, Wl, bl)


def _tc_pool(h, batch2d, Wp1, bp1, Wp2, bp2):
    def body(h_ref, b_ref, w1, b1, w2, b2, out_ref):
        bids = b_ref[...]
        gids = lax.broadcasted_iota(jnp.int32, (G, N), 0)
        onehot = (gids == bids).astype(jnp.float32)
        s = jnp.dot(onehot, h_ref[...], preferred_element_type=jnp.float32)
        c = jnp.sum(onehot, axis=1, keepdims=True)
        pooled = s / jnp.maximum(c, 1.0)
        z = jnp.dot(pooled, w1[...], preferred_element_type=jnp.float32) + b1[...]
        z = jnp.dot(z, w2[...], preferred_element_type=jnp.float32) + b2[...]
        z = z * jnp.minimum(c, 1.0)
        m = jnp.max(z, axis=1, keepdims=True)
        e = z - m
        lse = jnp.log(jnp.sum(jnp.exp(e), axis=1, keepdims=True))
        out_ref[...] = e - lse

    return pl.pallas_call(
        body,
        out_shape=jax.ShapeDtypeStruct((G, OUTD), jnp.float32),
    )(h, batch2d, Wp1, bp1, Wp2, bp2)


def kernel(x, edge_index, batch,
           Wl0, bl0, Wr0, br0,
           Wl1, bl1, Wr1, br1,
           Wl2, bl2, Wr2, br2,
           Wp1, bp1, Wp2, bp2):
    src = edge_index[0]
    dst = edge_index[1]
    zrow = jnp.zeros((N, D), jnp.float32)
    ones = jnp.ones((K, D), jnp.float32)

    src3 = src.reshape(NW * NSB, SUP, K)
    dst3 = dst.reshape(NW * NSB, SUP, K)

    cntp = _make_sc_degree()(dst, zrow, ones)
    part0 = _make_sc_segment_sum(D)(x, src3, dst3, zrow)
    h1, cnt = _tc_layer0(part0, cntp, x,
                         Wr0, br0.reshape(1, D), Wl0, bl0.reshape(1, D))
    part1 = _make_sc_segment_sum(D)(h1, src3, dst3, zrow)
    h2 = _tc_layer(part1, cnt, h1,
                   Wr1, br1.reshape(1, D), Wl1, bl1.reshape(1, D))
    part2 = _make_sc_segment_sum(D)(h2, src3, dst3, zrow)
    h3 = _tc_layer(part2, cnt, h2,
                   Wr2, br2.reshape(1, D), Wl2, bl2.reshape(1, D))
    return _tc_pool(h3, batch.reshape(1, N),
                    Wp1, bp1.reshape(1, HID), Wp2, bp2.reshape(1, OUTD))


# speedup vs baseline: 9.8185x; 1.1113x over previous
import functools

import jax
import jax.numpy as jnp
from jax import lax
from jax.experimental import pallas as pl
from jax.experimental.pallas import tpu as pltpu
from jax.experimental.pallas import tpu_sc as plsc

N = 10000
E = 320000
D = 128
HID = 128
OUTD = 64
G = 128

NC = 2
NS = 16
NW = NC * NS
EPT = E // NW
K = 80
NCHUNK = EPT // K
SUP = 25
NSB = NCHUNK // SUP
CW = 16
RPT = 632
RPT_LAST = N - 15 * RPT
RING = 4
DG = 2


def _zero_acc(sid, zrow_hbm, acc):
    rbase = pl.multiple_of(sid * RPT, 8)

    @pl.when(sid < NS - 1)
    def _():
        pltpu.sync_copy(zrow_hbm.at[pl.ds(rbase, RPT)],
                        acc.at[pl.ds(rbase, RPT)])

    @pl.when(sid == NS - 1)
    def _():
        pltpu.sync_copy(zrow_hbm.at[pl.ds((NS - 1) * RPT, RPT_LAST)],
                        acc.at[pl.ds((NS - 1) * RPT, RPT_LAST)])


def _copy_out(cid, sid, acc, out_hbm):
    rbase = pl.multiple_of(sid * RPT, 8)
    obase = pl.multiple_of(cid * N + rbase, 8)

    @pl.when(sid < NS - 1)
    def _():
        pltpu.sync_copy(acc.at[pl.ds(rbase, RPT)],
                        out_hbm.at[pl.ds(obase, RPT)])

    @pl.when(sid == NS - 1)
    def _():
        pltpu.sync_copy(
            acc.at[pl.ds((NS - 1) * RPT, RPT_LAST)],
            out_hbm.at[pl.ds(pl.multiple_of(cid * N + (NS - 1) * RPT, 8),
                             RPT_LAST)])


@functools.lru_cache(maxsize=None)
def _make_sc_degree():
    mesh = plsc.VectorSubcoreMesh(core_axis_name="c", subcore_axis_name="s")

    def body(dst_hbm, zrow_hbm, ones_hbm, out_hbm, dst_v, ones_v, acc, sem):
        cid = lax.axis_index("c")
        sid = lax.axis_index("s")
        wid = cid * NS + sid
        _zero_acc(sid, zrow_hbm, acc)
        pltpu.sync_copy(ones_hbm, ones_v)
        plsc.subcore_barrier()

        for sb in range(NSB):
            pltpu.sync_copy(dst_hbm.at[wid * NSB + sb], dst_v)
            for i in range(SUP):
                pltpu.async_copy(ones_v, acc.at[dst_v.at[i]], sem, add=True)
            for i in range(SUP):
                pltpu.make_async_copy(ones_v, acc.at[dst_v.at[i]],
                                      sem).wait()

        plsc.subcore_barrier()
        _copy_out(cid, sid, acc, out_hbm)

    return pl.kernel(
        body, mesh=mesh,
        out_type=jax.ShapeDtypeStruct((NC * N, D), jnp.float32),
        scratch_types=[
            pltpu.VMEM((SUP, K), jnp.int32),
            pltpu.VMEM((K, D), jnp.float32),
            pltpu.VMEM_SHARED((N, D), jnp.float32),
            pltpu.SemaphoreType.DMA,
        ])


def _tc_cnt(cntp):
    def body(c0, c1, out_ref):
        out_ref[...] = (c0[...] + c1[...])[:, :1]

    return pl.pallas_call(
        body,
        grid=(N // 1000,),
        in_specs=[
            pl.BlockSpec((1000, D), lambda i: (i, 0)),
            pl.BlockSpec((1000, D), lambda i: (i + N // 1000, 0)),
        ],
        out_specs=pl.BlockSpec((1000, 1), lambda i: (i, 0)),
        out_shape=jax.ShapeDtypeStruct((N, 1), jnp.float32),
    )(cntp, cntp)


@functools.lru_cache(maxsize=None)
def _make_sc_segment_sum(w):
    mesh = plsc.VectorSubcoreMesh(core_axis_name="c", subcore_axis_name="s")

    def body(h_hbm, src_hbm, dst_hbm, zrow_hbm, out_hbm,
             src_v, dst_v, rows_v, acc, *sems):
        cid = lax.axis_index("c")
        sid = lax.axis_index("s")
        wid = cid * NS + sid
        _zero_acc(sid, zrow_hbm, acc)
        plsc.subcore_barrier()

        sem_g = sems[:RING]
        sem_s = sems[RING:]

        def gather(i, sl):
            pltpu.async_copy(h_hbm.at[src_v.at[i]], rows_v.at[sl], sem_g[sl])

        def wait_gather(i, sl):
            pltpu.make_async_copy(h_hbm.at[src_v.at[i]], rows_v.at[sl],
                                  sem_g[sl]).wait()

        def scatter(i, sl):
            pltpu.async_copy(rows_v.at[sl], acc.at[dst_v.at[i]], sem_s[sl],
                             add=True)

        def wait_scatter(i, sl):
            pltpu.make_async_copy(rows_v.at[sl], acc.at[dst_v.at[i]],
                                  sem_s[sl]).wait()

        for sb in range(NSB):
            pltpu.sync_copy(src_hbm.at[wid * NSB + sb], src_v)
            pltpu.sync_copy(dst_hbm.at[wid * NSB + sb], dst_v)
            for g in range(DG):
                gather(g, g % RING)
            for i in range(SUP):
                wait_gather(i, i % RING)
                scatter(i, i % RING)
                g = i + DG
                if g < SUP:
                    if g >= RING:
                        wait_scatter(g - RING, g % RING)
                    gather(g, g % RING)
            for i in range(SUP - RING, SUP):
                wait_scatter(i, i % RING)

        plsc.subcore_barrier()
        _copy_out(cid, sid, acc, out_hbm)

    return pl.kernel(
        body, mesh=mesh,
        out_type=jax.ShapeDtypeStruct((NC * N, w), jnp.float32),
        scratch_types=[
            pltpu.VMEM((SUP, K), jnp.int32),
            pltpu.VMEM((SUP, K), jnp.int32),
            pltpu.VMEM((RING, K, w), jnp.float32),
            pltpu.VMEM_SHARED((N, w), jnp.float32),
        ] + [pltpu.SemaphoreType.DMA] * (2 * RING))


RB = 1000
_NB = N // RB


def _tc_layer(part, cnt, h, Wr, br, Wl, bl):
    def body(p0, p1, c_ref, h_ref, wr, brr, wl, blr, out_ref):
        p = p0[...] + p1[...]
        c = c_ref[...]
        inv = 1.0 / jnp.maximum(c, 1.0)
        ind = jnp.minimum(c, 1.0)
        agg = jnp.dot(p, wr[...], preferred_element_type=jnp.float32) * inv \
            + brr[...] * ind
        out = agg + jnp.dot(h_ref[...], wl[...],
                            preferred_element_type=jnp.float32) + blr[...]
        nrm = jnp.sqrt(jnp.sum(out * out, axis=1, keepdims=True))
        out = out / jnp.maximum(nrm, 1e-12)
        out_ref[...] = jnp.maximum(out, 0.0)

    return pl.pallas_call(
        body,
        grid=(_NB,),
        in_specs=[
            pl.BlockSpec((RB, D), lambda i: (i, 0)),
            pl.BlockSpec((RB, D), lambda i: (i + _NB, 0)),
            pl.BlockSpec((RB, 1), lambda i: (i, 0)),
            pl.BlockSpec((RB, D), lambda i: (i, 0)),
            pl.BlockSpec((D, D), lambda i: (0, 0)),
            pl.BlockSpec((1, D), lambda i: (0, 0)),
            pl.BlockSpec((D, D), lambda i: (0, 0)),
            pl.BlockSpec((1, D), lambda i: (0, 0)),
        ],
        out_specs=pl.BlockSpec((RB, D), lambda i: (i, 0)),
        out_shape=jax.ShapeDtypeStruct((N, D), jnp.float32),
    )(part, part, cnt, h, Wr, br, Wl, bl)


def _tc_pool(h, batch2d, Wp1, bp1, Wp2, bp2):
    def body(h_ref, b_ref, w1, b1, w2, b2, out_ref):
        bids = b_ref[...]
        gids = lax.broadcasted_iota(jnp.int32, (G, N), 0)
        onehot = (gids == bids).astype(jnp.float32)
        s = jnp.dot(onehot, h_ref[...], preferred_element_type=jnp.float32)
        c = jnp.sum(onehot, axis=1, keepdims=True)
        pooled = s / jnp.maximum(c, 1.0)
        z = jnp.dot(pooled, w1[...], preferred_element_type=jnp.float32) + b1[...]
        z = jnp.dot(z, w2[...], preferred_element_type=jnp.float32) + b2[...]
        z = z * jnp.minimum(c, 1.0)
        m = jnp.max(z, axis=1, keepdims=True)
        e = z - m
        lse = jnp.log(jnp.sum(jnp.exp(e), axis=1, keepdims=True))
        out_ref[...] = e - lse

    return pl.pallas_call(
        body,
        out_shape=jax.ShapeDtypeStruct((G, OUTD), jnp.float32),
    )(h, batch2d, Wp1, bp1, Wp2, bp2)


def kernel(x, edge_index, batch,
           Wl0, bl0, Wr0, br0,
           Wl1, bl1, Wr1, br1,
           Wl2, bl2, Wr2, br2,
           Wp1, bp1, Wp2, bp2):
    src = edge_index[0]
    dst = edge_index[1]
    zrow = jnp.zeros((N, D), jnp.float32)
    ones = jnp.ones((K, D), jnp.float32)

    src3 = src.reshape(NW * NSB, SUP, K)
    dst3 = dst.reshape(NW * NSB, SUP, K)

    cntp = _make_sc_degree()(dst3, zrow, ones)
    cnt = _tc_cnt(cntp)
    part0 = _make_sc_segment_sum(D)(x, src3, dst3, zrow)
    h1 = _tc_layer(part0, cnt, x,
                   Wr0, br0.reshape(1, D), Wl0, bl0.reshape(1, D))
    part1 = _make_sc_segment_sum(D)(h1, src3, dst3, zrow)
    h2 = _tc_layer(part1, cnt, h1,
                   Wr1, br1.reshape(1, D), Wl1, bl1.reshape(1, D))
    part2 = _make_sc_segment_sum(D)(h2, src3, dst3, zrow)
    h3 = _tc_layer(part2, cnt, h2,
                   Wr2, br2.reshape(1, D), Wl2, bl2.reshape(1, D))
    return _tc_pool(h3, batch.reshape(1, N),
                    Wp1, bp1.reshape(1, HID), Wp2, bp2.reshape(1, OUTD))


# speedup vs baseline: 11.4552x; 1.1667x over previous
import functools

import jax
import jax.numpy as jnp
from jax import lax
from jax.experimental import pallas as pl
from jax.experimental.pallas import tpu as pltpu
from jax.experimental.pallas import tpu_sc as plsc

N = 10000
E = 320000
D = 128
HID = 128
OUTD = 64
G = 128

NC = 2
NS = 16
NW = NC * NS
EPT = E // NW
K = 80
NCHUNK = EPT // K
SUP = 25
NSB = NCHUNK // SUP
CW = 16
RPT = 632
RPT_LAST = N - 15 * RPT
RING = 4
DG = 2


def _zero_acc(sid, zrow_hbm, acc):
    rbase = pl.multiple_of(sid * RPT, 8)

    @pl.when(sid < NS - 1)
    def _():
        pltpu.sync_copy(zrow_hbm.at[pl.ds(rbase, RPT)],
                        acc.at[pl.ds(rbase, RPT)])

    @pl.when(sid == NS - 1)
    def _():
        pltpu.sync_copy(zrow_hbm.at[pl.ds((NS - 1) * RPT, RPT_LAST)],
                        acc.at[pl.ds((NS - 1) * RPT, RPT_LAST)])


def _copy_out(cid, sid, acc, out_hbm):
    rbase = pl.multiple_of(sid * RPT, 8)
    obase = pl.multiple_of(cid * N + rbase, 8)

    @pl.when(sid < NS - 1)
    def _():
        pltpu.sync_copy(acc.at[pl.ds(rbase, RPT)],
                        out_hbm.at[pl.ds(obase, RPT)])

    @pl.when(sid == NS - 1)
    def _():
        pltpu.sync_copy(
            acc.at[pl.ds((NS - 1) * RPT, RPT_LAST)],
            out_hbm.at[pl.ds(pl.multiple_of(cid * N + (NS - 1) * RPT, 8),
                             RPT_LAST)])


NHI = 80
EHB = 32
EHW = E // EHB


def _tc_degree(dst2d):
    def body(d_ref, out_ref):
        i = pl.program_id(0)
        d = d_ref[0]
        hi = lax.shift_right_logical(d, 7)
        lo = lax.bitwise_and(d, 127)
        hi1h = (lax.broadcasted_iota(jnp.int32, (NHI, EHW), 0)
                == hi).astype(jnp.float32)
        lo1h = (lax.broadcasted_iota(jnp.int32, (D, EHW), 0)
                == lo).astype(jnp.float32)
        prod = lax.dot_general(lo1h, hi1h, (((1,), (1,)), ((), ())),
                               preferred_element_type=jnp.float32)

        @pl.when(i == 0)
        def _():
            out_ref[...] = jnp.zeros_like(out_ref)
        out_ref[...] += prod

    return pl.pallas_call(
        body,
        grid=(EHB,),
        in_specs=[pl.BlockSpec((1, 1, EHW), lambda i: (i, 0, 0))],
        out_specs=pl.BlockSpec((D, NHI), lambda i: (0, 0)),
        out_shape=jax.ShapeDtypeStruct((D, NHI), jnp.float32),
    )(dst2d)


@functools.lru_cache(maxsize=None)
def _make_sc_segment_sum(w):
    mesh = plsc.VectorSubcoreMesh(core_axis_name="c", subcore_axis_name="s")

    def body(h_hbm, src_hbm, dst_hbm, zrow_hbm, out_hbm,
             src_v, dst_v, rows_v, acc, *sems):
        cid = lax.axis_index("c")
        sid = lax.axis_index("s")
        wid = cid * NS + sid
        _zero_acc(sid, zrow_hbm, acc)
        plsc.subcore_barrier()

        sem_g = sems[:RING]
        sem_s = sems[RING:]

        def gather(i, sl):
            pltpu.async_copy(h_hbm.at[src_v.at[i]], rows_v.at[sl], sem_g[sl])

        def wait_gather(i, sl):
            pltpu.make_async_copy(h_hbm.at[src_v.at[i]], rows_v.at[sl],
                                  sem_g[sl]).wait()

        def scatter(i, sl):
            pltpu.async_copy(rows_v.at[sl], acc.at[dst_v.at[i]], sem_s[sl],
                             add=True)

        def wait_scatter(i, sl):
            pltpu.make_async_copy(rows_v.at[sl], acc.at[dst_v.at[i]],
                                  sem_s[sl]).wait()

        for sb in range(NSB):
            pltpu.sync_copy(src_hbm.at[wid * NSB + sb], src_v)
            pltpu.sync_copy(dst_hbm.at[wid * NSB + sb], dst_v)
            for g in range(DG):
                gather(g, g % RING)
            for i in range(SUP):
                wait_gather(i, i % RING)
                scatter(i, i % RING)
                g = i + DG
                if g < SUP:
                    if g >= RING:
                        wait_scatter(g - RING, g % RING)
                    gather(g, g % RING)
            for i in range(SUP - RING, SUP):
                wait_scatter(i, i % RING)

        plsc.subcore_barrier()
        _copy_out(cid, sid, acc, out_hbm)

    return pl.kernel(
        body, mesh=mesh,
        out_type=jax.ShapeDtypeStruct((NC * N, w), jnp.float32),
        scratch_types=[
            pltpu.VMEM((SUP, K), jnp.int32),
            pltpu.VMEM((SUP, K), jnp.int32),
            pltpu.VMEM((RING, K, w), jnp.float32),
            pltpu.VMEM_SHARED((N, w), jnp.float32),
        ] + [pltpu.SemaphoreType.DMA] * (2 * RING))


RB = 1000
_NB = N // RB


def _tc_layer(part, cnt, h, Wr, br, Wl, bl):
    def body(p0, p1, c_ref, h_ref, wr, brr, wl, blr, out_ref):
        p = p0[...] + p1[...]
        c = c_ref[...]
        inv = 1.0 / jnp.maximum(c, 1.0)
        ind = jnp.minimum(c, 1.0)
        agg = jnp.dot(p, wr[...], preferred_element_type=jnp.float32) * inv \
            + brr[...] * ind
        out = agg + jnp.dot(h_ref[...], wl[...],
                            preferred_element_type=jnp.float32) + blr[...]
        nrm = jnp.sqrt(jnp.sum(out * out, axis=1, keepdims=True))
        out = out / jnp.maximum(nrm, 1e-12)
        out_ref[...] = jnp.maximum(out, 0.0)

    return pl.pallas_call(
        body,
        grid=(_NB,),
        in_specs=[
            pl.BlockSpec((RB, D), lambda i: (i, 0)),
            pl.BlockSpec((RB, D), lambda i: (i + _NB, 0)),
            pl.BlockSpec((RB, 1), lambda i: (i, 0)),
            pl.BlockSpec((RB, D), lambda i: (i, 0)),
            pl.BlockSpec((D, D), lambda i: (0, 0)),
            pl.BlockSpec((1, D), lambda i: (0, 0)),
            pl.BlockSpec((D, D), lambda i: (0, 0)),
            pl.BlockSpec((1, D), lambda i: (0, 0)),
        ],
        out_specs=pl.BlockSpec((RB, D), lambda i: (i, 0)),
        out_shape=jax.ShapeDtypeStruct((N, D), jnp.float32),
    )(part, part, cnt, h, Wr, br, Wl, bl)


def _tc_pool(h, batch2d, Wp1, bp1, Wp2, bp2):
    def body(h_ref, b_ref, w1, b1, w2, b2, out_ref):
        bids = b_ref[...]
        gids = lax.broadcasted_iota(jnp.int32, (G, N), 0)
        onehot = (gids == bids).astype(jnp.float32)
        s = jnp.dot(onehot, h_ref[...], preferred_element_type=jnp.float32)
        c = jnp.sum(onehot, axis=1, keepdims=True)
        pooled = s / jnp.maximum(c, 1.0)
        z = jnp.dot(pooled, w1[...], preferred_element_type=jnp.float32) + b1[...]
        z = jnp.dot(z, w2[...], preferred_element_type=jnp.float32) + b2[...]
        z = z * jnp.minimum(c, 1.0)
        m = jnp.max(z, axis=1, keepdims=True)
        e = z - m
        lse = jnp.log(jnp.sum(jnp.exp(e), axis=1, keepdims=True))
        out_ref[...] = e - lse

    return pl.pallas_call(
        body,
        out_shape=jax.ShapeDtypeStruct((G, OUTD), jnp.float32),
    )(h, batch2d, Wp1, bp1, Wp2, bp2)


def kernel(x, edge_index, batch,
           Wl0, bl0, Wr0, br0,
           Wl1, bl1, Wr1, br1,
           Wl2, bl2, Wr2, br2,
           Wp1, bp1, Wp2, bp2):
    src = edge_index[0]
    dst = edge_index[1]
    zrow = jnp.zeros((N, D), jnp.float32)

    src3 = src.reshape(NW * NSB, SUP, K)
    dst3 = dst.reshape(NW * NSB, SUP, K)

    cnt2dT = _tc_degree(dst.reshape(EHB, 1, EHW))
    cnt = cnt2dT.T.reshape(NHI * D)[:N].reshape(N, 1)
    part0 = _make_sc_segment_sum(D)(x, src3, dst3, zrow)
    h1 = _tc_layer(part0, cnt, x,
                   Wr0, br0.reshape(1, D), Wl0, bl0.reshape(1, D))
    part1 = _make_sc_segment_sum(D)(h1, src3, dst3, zrow)
    h2 = _tc_layer(part1, cnt, h1,
                   Wr1, br1.reshape(1, D), Wl1, bl1.reshape(1, D))
    part2 = _make_sc_segment_sum(D)(h2, src3, dst3, zrow)
    h3 = _tc_layer(part2, cnt, h2,
                   Wr2, br2.reshape(1, D), Wl2, bl2.reshape(1, D))
    return _tc_pool(h3, batch.reshape(1, N),
                    Wp1, bp1.reshape(1, HID), Wp2, bp2.reshape(1, OUTD))


# speedup vs baseline: 12.2874x; 1.0726x over previous
import functools

import jax
import jax.numpy as jnp
from jax import lax
from jax.experimental import pallas as pl
from jax.experimental.pallas import tpu as pltpu
from jax.experimental.pallas import tpu_sc as plsc

N = 10000
E = 320000
D = 128
HID = 128
OUTD = 64
G = 128

NC = 2
NS = 16
NW = NC * NS
EPT = E // NW
K = 80
NCHUNK = EPT // K
SUP = 25
NSB = NCHUNK // SUP
CW = 16
RPT = 632
RPT_LAST = N - 15 * RPT
RING = 4
DG = 3


def _zero_acc(sid, zrow_hbm, acc):
    rbase = pl.multiple_of(sid * RPT, 8)

    @pl.when(sid < NS - 1)
    def _():
        pltpu.sync_copy(zrow_hbm.at[pl.ds(rbase, RPT)],
                        acc.at[pl.ds(rbase, RPT)])

    @pl.when(sid == NS - 1)
    def _():
        pltpu.sync_copy(zrow_hbm.at[pl.ds((NS - 1) * RPT, RPT_LAST)],
                        acc.at[pl.ds((NS - 1) * RPT, RPT_LAST)])


def _copy_out(cid, sid, acc, out_hbm):
    rbase = pl.multiple_of(sid * RPT, 8)
    obase = pl.multiple_of(cid * N + rbase, 8)

    @pl.when(sid < NS - 1)
    def _():
        pltpu.sync_copy(acc.at[pl.ds(rbase, RPT)],
                        out_hbm.at[pl.ds(obase, RPT)])

    @pl.when(sid == NS - 1)
    def _():
        pltpu.sync_copy(
            acc.at[pl.ds((NS - 1) * RPT, RPT_LAST)],
            out_hbm.at[pl.ds(pl.multiple_of(cid * N + (NS - 1) * RPT, 8),
                             RPT_LAST)])


NHI = 80
EHB = 32
EHW = E // EHB


def _tc_degree(dst2d):
    def body(d_ref, out_ref):
        i = pl.program_id(0)
        d = d_ref[0]
        hi = lax.shift_right_logical(d, 7)
        lo = lax.bitwise_and(d, 127)
        hi1h = (lax.broadcasted_iota(jnp.int32, (NHI, EHW), 0)
                == hi).astype(jnp.float32)
        lo1h = (lax.broadcasted_iota(jnp.int32, (D, EHW), 0)
                == lo).astype(jnp.float32)
        prod = lax.dot_general(lo1h, hi1h, (((1,), (1,)), ((), ())),
                               preferred_element_type=jnp.float32)

        @pl.when(i == 0)
        def _():
            out_ref[...] = jnp.zeros_like(out_ref)
        out_ref[...] += prod

    return pl.pallas_call(
        body,
        grid=(EHB,),
        in_specs=[pl.BlockSpec((1, 1, EHW), lambda i: (i, 0, 0))],
        out_specs=pl.BlockSpec((D, NHI), lambda i: (0, 0)),
        out_shape=jax.ShapeDtypeStruct((D, NHI), jnp.float32),
    )(dst2d)


@functools.lru_cache(maxsize=None)
def _make_sc_segment_sum(w):
    mesh = plsc.VectorSubcoreMesh(core_axis_name="c", subcore_axis_name="s")

    def body(h_hbm, src_hbm, dst_hbm, zrow_hbm, out_hbm,
             src_v, dst_v, rows_v, acc, *sems):
        cid = lax.axis_index("c")
        sid = lax.axis_index("s")
        wid = cid * NS + sid
        _zero_acc(sid, zrow_hbm, acc)
        plsc.subcore_barrier()

        sem_g = sems[:RING]
        sem_s = sems[RING:]

        def gather(i, sl):
            pltpu.async_copy(h_hbm.at[src_v.at[i]], rows_v.at[sl], sem_g[sl])

        def wait_gather(i, sl):
            pltpu.make_async_copy(h_hbm.at[src_v.at[i]], rows_v.at[sl],
                                  sem_g[sl]).wait()

        def scatter(i, sl):
            pltpu.async_copy(rows_v.at[sl], acc.at[dst_v.at[i]], sem_s[sl],
                             add=True)

        def wait_scatter(i, sl):
            pltpu.make_async_copy(rows_v.at[sl], acc.at[dst_v.at[i]],
                                  sem_s[sl]).wait()

        for sb in range(NSB):
            pltpu.sync_copy(src_hbm.at[wid * NSB + sb], src_v)
            pltpu.sync_copy(dst_hbm.at[wid * NSB + sb], dst_v)
            for g in range(DG):
                gather(g, g % RING)
            for i in range(SUP):
                wait_gather(i, i % RING)
                scatter(i, i % RING)
                g = i + DG
                if g < SUP:
                    if g >= RING:
                        wait_scatter(g - RING, g % RING)
                    gather(g, g % RING)
            for i in range(SUP - RING, SUP):
                wait_scatter(i, i % RING)

        plsc.subcore_barrier()
        _copy_out(cid, sid, acc, out_hbm)

    return pl.kernel(
        body, mesh=mesh,
        out_type=jax.ShapeDtypeStruct((NC * N, w), jnp.float32),
        scratch_types=[
            pltpu.VMEM((SUP, K), jnp.int32),
            pltpu.VMEM((SUP, K), jnp.int32),
            pltpu.VMEM((RING, K, w), jnp.float32),
            pltpu.VMEM_SHARED((N, w), jnp.float32),
        ] + [pltpu.SemaphoreType.DMA] * (2 * RING))


RB = 1000
_NB = N // RB


def _tc_layer(part, cnt, h, Wr, br, Wl, bl):
    def body(p0, p1, c_ref, h_ref, wr, brr, wl, blr, out_ref):
        p = p0[...] + p1[...]
        c = c_ref[...]
        inv = 1.0 / jnp.maximum(c, 1.0)
        ind = jnp.minimum(c, 1.0)
        agg = jnp.dot(p, wr[...], preferred_element_type=jnp.float32) * inv \
            + brr[...] * ind
        out = agg + jnp.dot(h_ref[...], wl[...],
                            preferred_element_type=jnp.float32) + blr[...]
        nrm = jnp.sqrt(jnp.sum(out * out, axis=1, keepdims=True))
        out = out / jnp.maximum(nrm, 1e-12)
        out_ref[...] = jnp.maximum(out, 0.0)

    return pl.pallas_call(
        body,
        grid=(_NB,),
        in_specs=[
            pl.BlockSpec((RB, D), lambda i: (i, 0)),
            pl.BlockSpec((RB, D), lambda i: (i + _NB, 0)),
            pl.BlockSpec((RB, 1), lambda i: (i, 0)),
            pl.BlockSpec((RB, D), lambda i: (i, 0)),
            pl.BlockSpec((D, D), lambda i: (0, 0)),
            pl.BlockSpec((1, D), lambda i: (0, 0)),
            pl.BlockSpec((D, D), lambda i: (0, 0)),
            pl.BlockSpec((1, D), lambda i: (0, 0)),
        ],
        out_specs=pl.BlockSpec((RB, D), lambda i: (i, 0)),
        out_shape=jax.ShapeDtypeStruct((N, D), jnp.float32),
    )(part, part, cnt, h, Wr, br, Wl, bl)


def _tc_pool(h, batch2d, Wp1, bp1, Wp2, bp2):
    def body(h_ref, b_ref, w1, b1, w2, b2, out_ref):
        bids = b_ref[...]
        gids = lax.broadcasted_iota(jnp.int32, (G, N), 0)
        onehot = (gids == bids).astype(jnp.float32)
        s = jnp.dot(onehot, h_ref[...], preferred_element_type=jnp.float32)
        c = jnp.sum(onehot, axis=1, keepdims=True)
        pooled = s / jnp.maximum(c, 1.0)
        z = jnp.dot(pooled, w1[...], preferred_element_type=jnp.float32) + b1[...]
        z = jnp.dot(z, w2[...], preferred_element_type=jnp.float32) + b2[...]
        z = z * jnp.minimum(c, 1.0)
        m = jnp.max(z, axis=1, keepdims=True)
        e = z - m
        lse = jnp.log(jnp.sum(jnp.exp(e), axis=1, keepdims=True))
        out_ref[...] = e - lse

    return pl.pallas_call(
        body,
        out_shape=jax.ShapeDtypeStruct((G, OUTD), jnp.float32),
    )(h, batch2d, Wp1, bp1, Wp2, bp2)


def kernel(x, edge_index, batch,
           Wl0, bl0, Wr0, br0,
           Wl1, bl1, Wr1, br1,
           Wl2, bl2, Wr2, br2,
           Wp1, bp1, Wp2, bp2):
    src = edge_index[0]
    dst = edge_index[1]
    zrow = jnp.zeros((N, D), jnp.float32)

    src3 = src.reshape(NW * NSB, SUP, K)
    dst3 = dst.reshape(NW * NSB, SUP, K)

    cnt2dT = _tc_degree(dst.reshape(EHB, 1, EHW))
    cnt = cnt2dT.T.reshape(NHI * D)[:N].reshape(N, 1)
    part0 = _make_sc_segment_sum(D)(x, src3, dst3, zrow)
    h1 = _tc_layer(part0, cnt, x,
                   Wr0, br0.reshape(1, D), Wl0, bl0.reshape(1, D))
    part1 = _make_sc_segment_sum(D)(h1, src3, dst3, zrow)
    h2 = _tc_layer(part1, cnt, h1,
                   Wr1, br1.reshape(1, D), Wl1, bl1.reshape(1, D))
    part2 = _make_sc_segment_sum(D)(h2, src3, dst3, zrow)
    h3 = _tc_layer(part2, cnt, h2,
                   Wr2, br2.reshape(1, D), Wl2, bl2.reshape(1, D))
    return _tc_pool(h3, batch.reshape(1, N),
                    Wp1, bp1.reshape(1, HID), Wp2, bp2.reshape(1, OUTD))


# speedup vs baseline: 12.8681x; 1.0473x over previous
import functools

import jax
import jax.numpy as jnp
from jax import lax
from jax.experimental import pallas as pl
from jax.experimental.pallas import tpu as pltpu
from jax.experimental.pallas import tpu_sc as plsc

N = 10000
E = 320000
D = 128
HID = 128
OUTD = 64
G = 128

NC = 2
NS = 16
NW = NC * NS
EPT = E // NW
K = 80
NCHUNK = EPT // K
SUP = 25
NSB = NCHUNK // SUP
CW = 16
RPT = 632
RPT_LAST = N - 15 * RPT
RING = 4
DG = 3
SUPD = 5
NBLK = NCHUNK // SUPD


def _zero_acc(sid, zrow_hbm, acc):
    rbase = pl.multiple_of(sid * RPT, 8)

    @pl.when(sid < NS - 1)
    def _():
        pltpu.sync_copy(zrow_hbm.at[pl.ds(rbase, RPT)],
                        acc.at[pl.ds(rbase, RPT)])

    @pl.when(sid == NS - 1)
    def _():
        pltpu.sync_copy(zrow_hbm.at[pl.ds((NS - 1) * RPT, RPT_LAST)],
                        acc.at[pl.ds((NS - 1) * RPT, RPT_LAST)])


def _copy_out(cid, sid, acc, out_hbm):
    rbase = pl.multiple_of(sid * RPT, 8)
    obase = pl.multiple_of(cid * N + rbase, 8)

    @pl.when(sid < NS - 1)
    def _():
        pltpu.sync_copy(acc.at[pl.ds(rbase, RPT)],
                        out_hbm.at[pl.ds(obase, RPT)])

    @pl.when(sid == NS - 1)
    def _():
        pltpu.sync_copy(
            acc.at[pl.ds((NS - 1) * RPT, RPT_LAST)],
            out_hbm.at[pl.ds(pl.multiple_of(cid * N + (NS - 1) * RPT, 8),
                             RPT_LAST)])


NHI = 80
EHB = 32
EHW = E // EHB


def _tc_degree(dst2d):
    def body(d_ref, out_ref):
        i = pl.program_id(0)
        d = d_ref[0]
        hi = lax.shift_right_logical(d, 7)
        lo = lax.bitwise_and(d, 127)
        hi1h = (lax.broadcasted_iota(jnp.int32, (NHI, EHW), 0)
                == hi).astype(jnp.float32)
        lo1h = (lax.broadcasted_iota(jnp.int32, (D, EHW), 0)
                == lo).astype(jnp.float32)
        prod = lax.dot_general(lo1h, hi1h, (((1,), (1,)), ((), ())),
                               preferred_element_type=jnp.float32)

        @pl.when(i == 0)
        def _():
            out_ref[...] = jnp.zeros_like(out_ref)
        out_ref[...] += prod

    return pl.pallas_call(
        body,
        grid=(EHB,),
        in_specs=[pl.BlockSpec((1, 1, EHW), lambda i: (i, 0, 0))],
        out_specs=pl.BlockSpec((D, NHI), lambda i: (0, 0)),
        out_shape=jax.ShapeDtypeStruct((D, NHI), jnp.float32),
    )(dst2d)


@functools.lru_cache(maxsize=None)
def _make_sc_segment_sum(w):
    mesh = plsc.VectorSubcoreMesh(core_axis_name="c", subcore_axis_name="s")

    def body(h_hbm, src_hbm, dst_hbm, zrow_hbm, out_hbm,
             src_v, dst_v, rows_v, acc, *sems):
        cid = lax.axis_index("c")
        sid = lax.axis_index("s")
        wid = cid * NS + sid
        _zero_acc(sid, zrow_hbm, acc)
        plsc.subcore_barrier()

        sem_g = sems[:RING]
        sem_s = sems[RING:2 * RING]
        sem_i = sems[2 * RING]

        def gather(t):
            pltpu.async_copy(h_hbm.at[src_v.at[t % SUP]],
                             rows_v.at[t % RING], sem_g[t % RING])

        def wait_gather(t):
            pltpu.make_async_copy(h_hbm.at[src_v.at[t % SUP]],
                                  rows_v.at[t % RING], sem_g[t % RING]).wait()

        def scatter(t):
            pltpu.async_copy(rows_v.at[t % RING],
                             acc.at[dst_v.at[(t // SUPD) % 2, t % SUPD]],
                             sem_s[t % RING], add=True)

        def wait_scatter(t):
            pltpu.make_async_copy(rows_v.at[t % RING],
                                  acc.at[dst_v.at[(t // SUPD) % 2, t % SUPD]],
                                  sem_s[t % RING]).wait()

        pltpu.sync_copy(src_hbm.at[wid * NSB], src_v)
        pltpu.sync_copy(dst_hbm.at[wid * NBLK], dst_v.at[0])
        for t in range(DG):
            gather(t)
        prewaited = set()
        for t in range(NCHUNK):
            b = t // SUPD
            if t % SUPD == 0 and b > 0:
                pltpu.make_async_copy(dst_hbm.at[wid * NBLK + b],
                                      dst_v.at[b % 2], sem_i).wait()
            if t not in prewaited:
                wait_gather(t)
            scatter(t)
            g = t + DG
            if g < NCHUNK:
                if g >= RING:
                    wait_scatter(g - RING)
                if g % SUP == 0:
                    for u in range(t + 1, g):
                        if u not in prewaited:
                            wait_gather(u)
                            prewaited.add(u)
                    pltpu.sync_copy(src_hbm.at[wid * NSB + g // SUP], src_v)
                gather(g)
            if t % SUPD == 0 and b + 1 < NBLK:
                pltpu.async_copy(dst_hbm.at[wid * NBLK + b + 1],
                                 dst_v.at[(b + 1) % 2], sem_i)
        for t in range(NCHUNK - RING, NCHUNK):
            wait_scatter(t)

        plsc.subcore_barrier()
        _copy_out(cid, sid, acc, out_hbm)

    return pl.kernel(
        body, mesh=mesh,
        out_type=jax.ShapeDtypeStruct((NC * N, w), jnp.float32),
        scratch_types=[
            pltpu.VMEM((SUP, K), jnp.int32),
            pltpu.VMEM((2, SUPD, K), jnp.int32),
            pltpu.VMEM((RING, K, w), jnp.float32),
            pltpu.VMEM_SHARED((N, w), jnp.float32),
        ] + [pltpu.SemaphoreType.DMA] * (2 * RING + 1))


RB = 1000
_NB = N // RB


def _tc_layer(part, cnt, h, Wr, br, Wl, bl):
    def body(p0, p1, c_ref, h_ref, wr, brr, wl, blr, out_ref):
        p = p0[...] + p1[...]
        c = c_ref[...]
        inv = 1.0 / jnp.maximum(c, 1.0)
        ind = jnp.minimum(c, 1.0)
        agg = jnp.dot(p, wr[...], preferred_element_type=jnp.float32) * inv \
            + brr[...] * ind
        out = agg + jnp.dot(h_ref[...], wl[...],
                            preferred_element_type=jnp.float32) + blr[...]
        nrm = jnp.sqrt(jnp.sum(out * out, axis=1, keepdims=True))
        out = out / jnp.maximum(nrm, 1e-12)
        out_ref[...] = jnp.maximum(out, 0.0)

    return pl.pallas_call(
        body,
        grid=(_NB,),
        in_specs=[
            pl.BlockSpec((RB, D), lambda i: (i, 0)),
            pl.BlockSpec((RB, D), lambda i: (i + _NB, 0)),
            pl.BlockSpec((RB, 1), lambda i: (i, 0)),
            pl.BlockSpec((RB, D), lambda i: (i, 0)),
            pl.BlockSpec((D, D), lambda i: (0, 0)),
            pl.BlockSpec((1, D), lambda i: (0, 0)),
            pl.BlockSpec((D, D), lambda i: (0, 0)),
            pl.BlockSpec((1, D), lambda i: (0, 0)),
        ],
        out_specs=pl.BlockSpec((RB, D), lambda i: (i, 0)),
        out_shape=jax.ShapeDtypeStruct((N, D), jnp.float32),
    )(part, part, cnt, h, Wr, br, Wl, bl)


def _tc_pool(h, batch2d, Wp1, bp1, Wp2, bp2):
    def body(h_ref, b_ref, w1, b1, w2, b2, out_ref):
        bids = b_ref[...]
        gids = lax.broadcasted_iota(jnp.int32, (G, N), 0)
        onehot = (gids == bids).astype(jnp.float32)
        s = jnp.dot(onehot, h_ref[...], preferred_element_type=jnp.float32)
        c = jnp.sum(onehot, axis=1, keepdims=True)
        pooled = s / jnp.maximum(c, 1.0)
        z = jnp.dot(pooled, w1[...], preferred_element_type=jnp.float32) + b1[...]
        z = jnp.dot(z, w2[...], preferred_element_type=jnp.float32) + b2[...]
        z = z * jnp.minimum(c, 1.0)
        m = jnp.max(z, axis=1, keepdims=True)
        e = z - m
        lse = jnp.log(jnp.sum(jnp.exp(e), axis=1, keepdims=True))
        out_ref[...] = e - lse

    return pl.pallas_call(
        body,
        out_shape=jax.ShapeDtypeStruct((G, OUTD), jnp.float32),
    )(h, batch2d, Wp1, bp1, Wp2, bp2)


def kernel(x, edge_index, batch,
           Wl0, bl0, Wr0, br0,
           Wl1, bl1, Wr1, br1,
           Wl2, bl2, Wr2, br2,
           Wp1, bp1, Wp2, bp2):
    src = edge_index[0]
    dst = edge_index[1]
    zrow = jnp.zeros((N, D), jnp.float32)

    src3 = src.reshape(NW * NSB, SUP, K)
    dst3 = dst.reshape(NW * NBLK, SUPD, K)

    cnt2dT = _tc_degree(dst.reshape(EHB, 1, EHW))
    cnt = cnt2dT.T.reshape(NHI * D)[:N].reshape(N, 1)
    part0 = _make_sc_segment_sum(D)(x, src3, dst3, zrow)
    h1 = _tc_layer(part0, cnt, x,
                   Wr0, br0.reshape(1, D), Wl0, bl0.reshape(1, D))
    part1 = _make_sc_segment_sum(D)(h1, src3, dst3, zrow)
    h2 = _tc_layer(part1, cnt, h1,
                   Wr1, br1.reshape(1, D), Wl1, bl1.reshape(1, D))
    part2 = _make_sc_segment_sum(D)(h2, src3, dst3, zrow)
    h3 = _tc_layer(part2, cnt, h2,
                   Wr2, br2.reshape(1, D), Wl2, bl2.reshape(1, D))
    return _tc_pool(h3, batch.reshape(1, N),
                    Wp1, bp1.reshape(1, HID), Wp2, bp2.reshape(1, OUTD))
